# Initial kernel scaffold; baseline (speedup 1.0000x reference)
import jax
import jax.numpy as jnp
from jax.experimental import pallas as pl


def kernel(condition, centers0, gamma0, W0, b0, centers1, gamma1, W1, b1, emb0, emb1):
    raise NotImplementedError("write your pallas kernel here")


# speedup vs baseline: 4.8327x; 4.8327x over previous
import jax
import jax.numpy as jnp
from jax.experimental import pallas as pl

_BLOCK = 2048
_D = 128


def _fused_body(cond_ref, w_ref, b_ref, consts_ref, out_ref):
    cond = cond_ref[...]
    x1 = cond[:, 1:2]
    x3 = cond[:, 3:4]
    idx0 = cond[:, 0:1].astype(jnp.int32)
    idx1 = cond[:, 2:3].astype(jnp.int32)
    blk = cond.shape[0]
    lane = jax.lax.broadcasted_iota(jnp.int32, (blk, _D), 1)
    ccat = consts_ref[0:1, :]
    gcat = consts_ref[1:2, :]
    xsel = jnp.where(lane < 10, x1, x3)
    rbf = jnp.exp(-gcat * jnp.square(xsel - ccat))
    target = jnp.where(lane < 117, idx0 + 110, idx1 + 117)
    onehot = (lane == target).astype(jnp.float32)
    feats = jnp.where(lane < 110, rbf, onehot)
    out_ref[...] = jax.lax.dot_general(
        feats, w_ref[...], (((1,), (0,)), ((), ())),
        preferred_element_type=jnp.float32,
        precision=jax.lax.Precision.HIGHEST) + b_ref[0:1, :]


def kernel(condition, centers0, gamma0, W0, b0, centers1, gamma1, W1, b1,
           emb0, emb1):
    n0 = centers0.shape[0]
    n1 = centers1.shape[0]
    pad = _D - n0 - n1 - emb0.shape[0] - emb1.shape[0]
    w_cat = jnp.concatenate([W0, W1, emb0, emb1], axis=0)
    bias = (b0 + b1).reshape(1, _D)
    zeros_pad = jnp.zeros((_D - n0 - n1,), jnp.float32)
    crow = jnp.concatenate([centers0, centers1, zeros_pad])
    grow = jnp.concatenate([jnp.broadcast_to(gamma0, (n0,)),
                            jnp.broadcast_to(gamma1, (n1,)), zeros_pad])
    consts = jnp.stack([crow, grow])
    del pad
    batch = condition.shape[0]
    return pl.pallas_call(
        _fused_body,
        grid=(batch // _BLOCK,),
        in_specs=[
            pl.BlockSpec((_BLOCK, 4), lambda i: (i, 0)),
            pl.BlockSpec((_D, _D), lambda i: (0, 0)),
            pl.BlockSpec((1, _D), lambda i: (0, 0)),
            pl.BlockSpec((2, _D), lambda i: (0, 0)),
        ],
        out_specs=pl.BlockSpec((_BLOCK, _D), lambda i: (i, 0)),
        out_shape=jax.ShapeDtypeStruct((batch, _D), jnp.float32),
    )(condition, w_cat, bias, consts)
